# baseline (speedup 1.0000x reference)
import functools

import jax
import jax.numpy as jnp
from jax import lax
from jax.experimental import pallas as pl
from jax.experimental.pallas import tpu as pltpu
from jax.experimental.pallas import tpu_sc as plsc

VOCAB = 1000000
DIM = 64
ROWS = 16384
COLS = 50
NC, NS = 2, 16
NW = NC * NS
ROWS_W = ROWS // NW
RCHUNK = 8
N_CHUNKS = ROWS_W // RCHUNK

_MESH = plsc.VectorSubcoreMesh(
    core_axis_name="c", subcore_axis_name="s", num_cores=NC, num_subcores=NS
)


@functools.partial(
    pl.kernel,
    out_type=jax.ShapeDtypeStruct((ROWS, COLS, DIM), jnp.float32),
    mesh=_MESH,
    scratch_types=[
        pltpu.VMEM((RCHUNK, COLS), jnp.int32),
        pltpu.VMEM((RCHUNK, COLS), jnp.int32),
        pltpu.VMEM((RCHUNK, COLS, DIM), jnp.float32),
        pltpu.VMEM((RCHUNK, COLS, DIM), jnp.float32),
        pltpu.SemaphoreType.DMA,
        pltpu.SemaphoreType.DMA,
        pltpu.SemaphoreType.DMA,
        pltpu.SemaphoreType.DMA,
    ],
    compiler_params=pltpu.CompilerParams(use_tc_tiling_on_sc=False),
)
def _gather_kernel(x_hbm, w_hbm, out_hbm, idx0, idx1, rows0, rows1,
                   gsem0, gsem1, ssem0, ssem1):
    wid = lax.axis_index("s") * NC + lax.axis_index("c")
    xrow0 = wid * ROWS_W

    def idx_load(g, idx_v):
        pltpu.sync_copy(x_hbm.at[pl.ds(xrow0 + g * RCHUNK, RCHUNK)], idx_v)

    def fire_g(idx_v, rows_v, sem):
        for r in range(RCHUNK):
            pltpu.async_copy(w_hbm.at[idx_v.at[r]], rows_v.at[r], sem)

    def wait_g(rows_v, sem):
        pltpu.make_async_copy(out_hbm.at[pl.ds(0, RCHUNK)], rows_v, sem).wait()

    def fire_s(g, rows_v, sem):
        pltpu.async_copy(
            rows_v, out_hbm.at[pl.ds(xrow0 + g * RCHUNK, RCHUNK)], sem
        )

    def wait_s(rows_v, sem):
        pltpu.make_async_copy(rows_v, out_hbm.at[pl.ds(0, RCHUNK)], sem).wait()

    idx_load(0, idx0)
    fire_g(idx0, rows0, gsem0)

    def pair(i, _):
        j = i * 2

        @pl.when(i > 0)
        def _():
            wait_s(rows1, ssem1)

        idx_load(j + 1, idx1)
        fire_g(idx1, rows1, gsem1)

        wait_g(rows0, gsem0)
        fire_s(j, rows0, ssem0)

        @pl.when(j + 2 < N_CHUNKS)
        def _():
            wait_s(rows0, ssem0)
            idx_load(j + 2, idx0)
            fire_g(idx0, rows0, gsem0)

        wait_g(rows1, gsem1)
        fire_s(j + 1, rows1, ssem1)
        return 0

    lax.fori_loop(0, N_CHUNKS // 2, pair, 0)

    wait_s(rows0, ssem0)
    wait_s(rows1, ssem1)


def kernel(x, weight):
    return _gather_kernel(x.astype(jnp.int32), weight)
